# baseline (speedup 1.0000x reference)
import functools

import jax
import jax.numpy as jnp
from jax import lax
from jax.experimental import pallas as pl
from jax.experimental.pallas import tpu as pltpu, tpu_sc as plsc

VOCAB = 1_000_000
D = 32
NC, NS = 2, 16
NW = NC * NS

BATCH, HIST = 16384, 50
BL = 128
ROWS_PER_W = BATCH // NW
BLOCKS_PER_W = ROWS_PER_W // BL
UNITS_PER_W = BLOCKS_PER_W * HIST


def _gather_body(table_hbm, idx_hbm, out_hbm, x_v, xt_v, rows_v, trans_v,
                 gsem, osem):
    wid = lax.axis_index("s") * NC + lax.axis_index("c")
    tc0 = wid * BLOCKS_PER_W
    b0w = tc0 * BL

    def unit_coords(u):
        return lax.div(u, BLOCKS_PER_W), lax.rem(u, BLOCKS_PER_W)

    def fire_gather(u, buf):
        h, tcl = unit_coords(u)
        pltpu.async_copy(
            table_hbm.at[xt_v.at[h, pl.ds(tcl * BL, BL)]],
            rows_v.at[buf],
            gsem,
        )

    def wait_gather(buf):
        pltpu.make_async_copy(
            table_hbm.at[pl.ds(0, BL)], rows_v.at[buf], gsem,
        ).wait()

    def out_slice(u):
        h, tcl = unit_coords(u)
        return out_hbm.at[h, pl.ds(0, 4), tc0 + tcl]

    def fire_out(u, buf):
        pltpu.async_copy(trans_v.at[buf], out_slice(u), osem)

    def wait_out(u, buf):
        pltpu.make_async_copy(trans_v.at[buf], out_slice(u), osem).wait()

    iota = lax.iota(jnp.int32, 16)

    def transpose_x():
        for h in range(HIST):
            colv = jnp.full((16,), h, jnp.int32)

            def step(k, _):
                rowv = iota + 16 * k
                v = plsc.load_gather(x_v, [rowv, colv])
                xt_v[h, pl.ds(pl.multiple_of(16 * k, 16), 16)] = v
                return ()

            lax.fori_loop(0, ROWS_PER_W // 16, step, (), unroll=False)

    def transpose_rows(buf):
        for e in range(D):
            colv = jnp.full((16,), e, jnp.int32)
            for k in range(BL // 16):
                rowv = iota + (16 * k)
                v = plsc.load_gather(rows_v.at[buf], [rowv, colv])
                trans_v[buf, e // 8, e % 8, pl.ds(16 * k, 16)] = v

    pltpu.sync_copy(idx_hbm.at[pl.ds(b0w, ROWS_PER_W)], x_v)
    transpose_x()
    fire_gather(0, 0)

    def body(g, _):
        for b in (0, 1):
            u = 2 * g + b
            wait_gather(b)

            def _fire_next():
                fire_gather(u + 1, 1 - b)
            if b == 0:
                _fire_next()
            else:
                pl.when(g + 1 < UNITS_PER_W // 2)(_fire_next)

            @pl.when(u >= 2)
            def _():
                wait_out(u - 2, b)

            transpose_rows(b)
            fire_out(u, b)
        return ()

    lax.fori_loop(0, UNITS_PER_W // 2, body, (), unroll=False)
    wait_out(UNITS_PER_W - 2, 0)
    wait_out(UNITS_PER_W - 1, 1)


@functools.partial(jax.jit, static_argnames=())
def kernel(x, embeddings):
    mesh = plsc.VectorSubcoreMesh(core_axis_name="c", subcore_axis_name="s")
    run = pl.kernel(
        _gather_body,
        out_type=jax.ShapeDtypeStruct((HIST, 4, 128, 8, BL), jnp.float32),
        mesh=mesh,
        scratch_types=[
            pltpu.VMEM((ROWS_PER_W, HIST), jnp.int32),
            pltpu.VMEM((HIST, ROWS_PER_W), jnp.int32),
            pltpu.VMEM((2, BL, D), jnp.float32),
            pltpu.VMEM((2, 4, 8, BL), jnp.float32),
            pltpu.SemaphoreType.DMA,
            pltpu.SemaphoreType.DMA,
        ],
        compiler_params=pltpu.CompilerParams(
            use_tc_tiling_on_sc=False, needs_layout_passes=False
        ),
    )
    out5 = run(embeddings, x)
    return out5.transpose(2, 4, 0, 1, 3).reshape(BATCH, HIST, D)


# speedup vs baseline: 1.1118x; 1.1118x over previous
import functools

import jax
import jax.numpy as jnp
from jax import lax
from jax.experimental import pallas as pl
from jax.experimental.pallas import tpu as pltpu, tpu_sc as plsc

VOCAB = 1_000_000
D = 32
NC, NS = 2, 16
NW = NC * NS

BATCH, HIST = 16384, 50
BL = 128
ROWS_PER_W = BATCH // NW
BLOCKS_PER_W = ROWS_PER_W // BL
UNITS_PER_W = BLOCKS_PER_W * HIST


def _gather_body(table_hbm, idx_hbm, out_hbm, x_v, xt_v, rows_v, trans_v,
                 gsem, osem):
    wid = lax.axis_index("s") * NC + lax.axis_index("c")
    tc0 = wid * BLOCKS_PER_W
    b0w = tc0 * BL

    def unit_coords(u):
        return lax.div(u, BLOCKS_PER_W), lax.rem(u, BLOCKS_PER_W)

    def fire_gather(u, buf):
        h, tcl = unit_coords(u)
        pltpu.async_copy(
            table_hbm.at[xt_v.at[h, pl.ds(tcl * BL, BL)]],
            rows_v.at[buf],
            gsem,
        )

    def wait_gather(buf):
        pltpu.make_async_copy(
            table_hbm.at[pl.ds(0, BL)], rows_v.at[buf], gsem,
        ).wait()

    def out_slice(u):
        h, tcl = unit_coords(u)
        return out_hbm.at[h, pl.ds(0, 4), tc0 + tcl]

    def fire_out(u, buf):
        pltpu.async_copy(trans_v.at[buf], out_slice(u), osem)

    def wait_out(u, buf):
        pltpu.make_async_copy(trans_v.at[buf], out_slice(u), osem).wait()

    iota = lax.iota(jnp.int32, 16)

    def transpose_x():
        for h in range(HIST):
            colv = jnp.full((16,), h, jnp.int32)

            def step(k, _):
                rowv = iota + 16 * k
                v = plsc.load_gather(x_v, [rowv, colv])
                xt_v[h, pl.ds(pl.multiple_of(16 * k, 16), 16)] = v
                return ()

            lax.fori_loop(0, ROWS_PER_W // 16, step, (), unroll=False)

    rowvs = [iota + 16 * k for k in range(BL // 16)]

    def transpose_rows(buf):
        for e in range(D):
            colv = jnp.full((16,), e, jnp.int32)
            vs = [
                plsc.load_gather(rows_v.at[buf], [rowvs[k], colv])
                for k in range(BL // 16)
            ]
            for k in range(BL // 16):
                trans_v[buf, e // 8, e % 8, pl.ds(16 * k, 16)] = vs[k]

    NB = 4

    pltpu.sync_copy(idx_hbm.at[pl.ds(b0w, ROWS_PER_W)], x_v)
    transpose_x()
    for u in range(NB - 1):
        fire_gather(u, u)

    def body(g, _):
        for b in range(NB):
            u = NB * g + b
            wait_gather(b)

            @pl.when(u + NB - 1 < UNITS_PER_W)
            def _():
                fire_gather(u + NB - 1, (b + NB - 1) % NB)

            @pl.when(u >= NB)
            def _():
                wait_out(u - NB, b)

            transpose_rows(b)
            fire_out(u, b)
        return ()

    lax.fori_loop(0, UNITS_PER_W // NB, body, (), unroll=False)
    for b in range(NB):
        wait_out(UNITS_PER_W - NB + b, b)


@functools.partial(jax.jit, static_argnames=())
def kernel(x, embeddings):
    mesh = plsc.VectorSubcoreMesh(core_axis_name="c", subcore_axis_name="s")
    run = pl.kernel(
        _gather_body,
        out_type=jax.ShapeDtypeStruct((HIST, 4, 128, 8, BL), jnp.float32),
        mesh=mesh,
        scratch_types=[
            pltpu.VMEM((ROWS_PER_W, HIST), jnp.int32),
            pltpu.VMEM((HIST, ROWS_PER_W), jnp.int32),
            pltpu.VMEM((4, BL, D), jnp.float32),
            pltpu.VMEM((4, 4, 8, BL), jnp.float32),
            pltpu.SemaphoreType.DMA,
            pltpu.SemaphoreType.DMA,
        ],
        compiler_params=pltpu.CompilerParams(
            use_tc_tiling_on_sc=False, needs_layout_passes=False
        ),
    )
    out5 = run(embeddings, x)
    return out5.transpose(2, 4, 0, 1, 3).reshape(BATCH, HIST, D)


# speedup vs baseline: 1.3708x; 1.2330x over previous
import functools

import jax
import jax.numpy as jnp
from jax import lax
from jax.experimental import pallas as pl
from jax.experimental.pallas import tpu as pltpu, tpu_sc as plsc

VOCAB = 1_000_000
D = 32
NC, NS = 2, 16
NW = NC * NS

BATCH, HIST = 16384, 50
BL = 128
ROWS_PER_W = BATCH // NW
BLOCKS_PER_W = ROWS_PER_W // BL
UNITS_PER_W = BLOCKS_PER_W * HIST


def _gather_body(table_hbm, idx_hbm, out_hbm, x_v, xt_v, rows_v, rows_p,
                 trans_v, gsem, osem):
    wid = lax.axis_index("s") * NC + lax.axis_index("c")
    tc0 = wid * BLOCKS_PER_W
    b0w = tc0 * BL

    def unit_coords(u):
        return lax.div(u, BLOCKS_PER_W), lax.rem(u, BLOCKS_PER_W)

    def fire_gather(u, buf):
        h, tcl = unit_coords(u)
        pltpu.async_copy(
            table_hbm.at[xt_v.at[h, pl.ds(tcl * BL, BL)]],
            rows_v.at[buf],
            gsem,
        )

    def wait_gather(buf):
        pltpu.make_async_copy(
            table_hbm.at[pl.ds(0, BL)], rows_v.at[buf], gsem,
        ).wait()

    def out_slice(u):
        h, tcl = unit_coords(u)
        return out_hbm.at[h, pl.ds(0, 4), tc0 + tcl]

    def fire_out(u, buf):
        pltpu.async_copy(trans_v.at[buf], out_slice(u), osem)

    def wait_out(u, buf):
        pltpu.make_async_copy(trans_v.at[buf], out_slice(u), osem).wait()

    iota = lax.iota(jnp.int32, 16)

    def transpose_x():
        for h in range(HIST):
            colv = jnp.full((16,), h, jnp.int32)

            def step(k, _):
                rowv = iota + 16 * k
                v = plsc.load_gather(x_v, [rowv, colv])
                xt_v[h, pl.ds(pl.multiple_of(16 * k, 16), 16)] = v
                return ()

            lax.fori_loop(0, ROWS_PER_W // 16, step, (), unroll=False)

    PS = D + 1
    iota33 = [(iota + 16 * k) * PS for k in range(BL // 16)]

    def transpose_rows(buf):
        def stage(b, _):
            for q in range(2):
                v = rows_v[buf, b, pl.ds(16 * q, 16)]
                plsc.store_scatter(rows_p, [iota + (b * PS + 16 * q)], v)
            return ()

        lax.fori_loop(0, BL, stage, (), unroll=8)

        for e in range(D):
            vs = [
                plsc.load_gather(rows_p, [iota33[k] + e])
                for k in range(BL // 16)
            ]
            for k in range(BL // 16):
                trans_v[buf, e // 8, e % 8, pl.ds(16 * k, 16)] = vs[k]

    NB = 4

    pltpu.sync_copy(idx_hbm.at[pl.ds(b0w, ROWS_PER_W)], x_v)
    transpose_x()
    for u in range(NB - 1):
        fire_gather(u, u)

    def body(g, _):
        for b in range(NB):
            u = NB * g + b
            wait_gather(b)

            @pl.when(u + NB - 1 < UNITS_PER_W)
            def _():
                fire_gather(u + NB - 1, (b + NB - 1) % NB)

            @pl.when(u >= NB)
            def _():
                wait_out(u - NB, b)

            transpose_rows(b)
            fire_out(u, b)
        return ()

    lax.fori_loop(0, UNITS_PER_W // NB, body, (), unroll=False)
    for b in range(NB):
        wait_out(UNITS_PER_W - NB + b, b)


@functools.partial(jax.jit, static_argnames=())
def kernel(x, embeddings):
    mesh = plsc.VectorSubcoreMesh(core_axis_name="c", subcore_axis_name="s")
    run = pl.kernel(
        _gather_body,
        out_type=jax.ShapeDtypeStruct((HIST, 4, 128, 8, BL), jnp.float32),
        mesh=mesh,
        scratch_types=[
            pltpu.VMEM((ROWS_PER_W, HIST), jnp.int32),
            pltpu.VMEM((HIST, ROWS_PER_W), jnp.int32),
            pltpu.VMEM((4, BL, D), jnp.float32),
            pltpu.VMEM((BL * (D + 1),), jnp.float32),
            pltpu.VMEM((4, 4, 8, BL), jnp.float32),
            pltpu.SemaphoreType.DMA,
            pltpu.SemaphoreType.DMA,
        ],
        compiler_params=pltpu.CompilerParams(
            use_tc_tiling_on_sc=False, needs_layout_passes=False
        ),
    )
    out5 = run(embeddings, x)
    return out5.transpose(2, 4, 0, 1, 3).reshape(BATCH, HIST, D)
